# baseline (speedup 1.0000x reference)
import dataclasses
import functools

import jax
import jax.numpy as jnp
from jax import lax
from jax.experimental import pallas as pl
from jax.experimental.pallas import tpu as pltpu
from jax.experimental.pallas import tpu_sc as plsc

N = 10000
E = 320000
D = 128
NP = 10240
NW = 32
EW = E // NW

_sc_mesh = plsc.VectorSubcoreMesh(core_axis_name="c", subcore_axis_name="s")

_sc_params = pltpu.CompilerParams()
if "needs_layout_passes" in pltpu.CompilerParams.__dataclass_fields__:
    _sc_params = dataclasses.replace(_sc_params, needs_layout_passes=False)


def _pre_body(xl_ref, w_ref, att_ref, h_ref, st_ref):
    w = w_ref[...]
    att = att_ref[...]
    vs = jnp.dot(w.T, att[0, :D], preferred_element_type=jnp.float32)
    vd = jnp.dot(w.T, att[0, D:2 * D], preferred_element_type=jnp.float32)
    xl = xl_ref[...]
    h_ref[...] = jnp.dot(xl, w.T, preferred_element_type=jnp.float32)
    st_ref[...] = jnp.dot(xl, jnp.stack([vs, vd], axis=1),
                          preferred_element_type=jnp.float32)


@jax.jit
def _pre(xl, W, att):
    return pl.pallas_call(
        _pre_body,
        out_shape=(jax.ShapeDtypeStruct((N, D), jnp.float32),
                   jax.ShapeDtypeStruct((N, 2), jnp.float32)),
    )(xl, W, att)


def _u_body(wr1_ref, att1_ref, wr2_ref, att2_ref, rh_ref, u_ref):
    vr1 = jnp.dot(wr1_ref[...].T, att1_ref[0, 2 * D:],
                  preferred_element_type=jnp.float32)
    vr2 = jnp.dot(wr2_ref[...].T, att2_ref[0, 2 * D:],
                  preferred_element_type=jnp.float32)
    u_ref[...] = jnp.dot(rh_ref[...], jnp.stack([vr1, vr2], axis=1),
                         preferred_element_type=jnp.float32)


@jax.jit
def _u_pass(r_h, Wr1, att1, Wr2, att2):
    blk = 16000
    return pl.pallas_call(
        _u_body,
        grid=(E // blk,),
        in_specs=[
            pl.BlockSpec((D, D), lambda i: (0, 0)),
            pl.BlockSpec((1, 3 * D), lambda i: (0, 0)),
            pl.BlockSpec((D, D), lambda i: (0, 0)),
            pl.BlockSpec((1, 3 * D), lambda i: (0, 0)),
            pl.BlockSpec((blk, D), lambda i: (i, 0)),
        ],
        out_specs=pl.BlockSpec((blk, 2), lambda i: (i, 0)),
        out_shape=jax.ShapeDtypeStruct((E, 2), jnp.float32),
    )(Wr1, att1, Wr2, att2, r_h)


def _post_body(p_ref, q_ref, h_ref, den_ref, w_ref, wr_ref, lw_ref, o_ref):
    den = jnp.sum(den_ref[...], axis=0)[:, None]
    safe = jnp.where(den > 0.0, den, 1.0)
    h = h_ref[...]
    agg = (jnp.dot(p_ref[...], w_ref[...].T, preferred_element_type=jnp.float32)
           + jnp.dot(q_ref[...], wr_ref[...].T, preferred_element_type=jnp.float32)) / safe
    out = jnp.where(den > 0.0,
                    agg + jnp.dot(h, lw_ref[...], preferred_element_type=jnp.float32),
                    h)
    o_ref[...] = jnp.maximum(out, 0.0)


@jax.jit
def _post(P, Q, h, den16, W, Wr, Lw):
    return pl.pallas_call(
        _post_body,
        out_shape=jax.ShapeDtypeStruct((N, D), jnp.float32),
    )(P, Q, h, den16, W, Wr, Lw)


@functools.partial(
    pl.kernel,
    out_type=[jax.ShapeDtypeStruct((E,), jnp.float32),
              jax.ShapeDtypeStruct((NW, 16), jnp.float32)],
    mesh=_sc_mesh,
    scratch_types=[
        pltpu.VMEM((NP,), jnp.float32),
        pltpu.VMEM((NP,), jnp.float32),
        pltpu.VMEM((EW,), jnp.int32),
        pltpu.VMEM((EW,), jnp.int32),
        pltpu.VMEM((EW,), jnp.float32),
        pltpu.VMEM((EW,), jnp.float32),
        pltpu.VMEM((16,), jnp.float32),
        pltpu.SemaphoreType.DMA,
    ],
    compiler_params=_sc_params)
def _s1(s_hbm, t_hbm, src_hbm, dst_hbm, u_hbm, e_hbm, m_hbm,
        s_v, t_v, src_v, dst_v, u_v, e_v, mx_v, sem):
    wid = lax.axis_index("s") * 2 + lax.axis_index("c")
    base = wid * EW
    pltpu.async_copy(s_hbm, s_v, sem).wait()
    pltpu.async_copy(t_hbm, t_v, sem).wait()
    pltpu.async_copy(src_hbm.at[pl.ds(base, EW)], src_v, sem).wait()
    pltpu.async_copy(dst_hbm.at[pl.ds(base, EW)], dst_v, sem).wait()
    pltpu.async_copy(u_hbm.at[pl.ds(base, EW)], u_v, sem).wait()
    mx_v[...] = jnp.full((16,), -3.0e38, jnp.float32)

    @pl.loop(0, EW, step=16)
    def _(i):
        sl = pl.ds(i, 16)
        e16 = (plsc.load_gather(s_v, [src_v[sl]])
               + plsc.load_gather(t_v, [dst_v[sl]])
               + u_v[sl])
        e16 = jnp.where(e16 >= 0.0, e16, 0.01 * e16)
        e_v[sl] = e16
        mx_v[...] = jnp.maximum(mx_v[...], e16)

    pltpu.async_copy(e_v, e_hbm.at[pl.ds(base, EW)], sem).wait()
    pltpu.async_copy(mx_v, m_hbm.at[wid], sem).wait()


NT = 16
NE = E // NT
CH = 80
NR = NP // NT
NCH = NE // CH
GSZ = 3 * CH
NG = 82


@functools.partial(
    pl.kernel,
    out_type=[jax.ShapeDtypeStruct((NP, D), jnp.float32),
              jax.ShapeDtypeStruct((NP, D), jnp.float32),
              jax.ShapeDtypeStruct((NT, NP), jnp.float32)],
    mesh=_sc_mesh,
    scratch_types=(
        [pltpu.VMEM_SHARED((NP, D), jnp.float32)]
        + [pltpu.VMEM((GSZ,), jnp.int32) for _ in range(2)]
        + [pltpu.VMEM((GSZ,), jnp.int32) for _ in range(2)]
        + [pltpu.VMEM((GSZ,), jnp.float32) for _ in range(2)]
        + [pltpu.VMEM((CH, D), jnp.float32) for _ in range(3)]
        + [pltpu.VMEM((CH,), jnp.int32) for _ in range(3)]
        + [pltpu.VMEM((CH,), jnp.int32) for _ in range(3)]
        + [pltpu.VMEM((NP,), jnp.float32),
           pltpu.VMEM((NW, 16), jnp.float32)]
        + [pltpu.SemaphoreType.DMA]
        + [pltpu.SemaphoreType.DMA for _ in range(2)]
        + [pltpu.SemaphoreType.DMA for _ in range(6)]
    ),
    compiler_params=_sc_params)
def _s2(e_hbm, m_hbm, src_hbm, dst_hbm, xl_hbm, rh_hbm,
        p_hbm, q_hbm, den_hbm,
        acc_sh, srcA, srcB, dstA, dstB, eA, eB,
        rows0, rows1, rows2,
        sidx0, sidx1, sidx2,
        didx0, didx1, didx2,
        den_acc, m_v, sem, sst0, sst1,
        sg0, sg1, sg2, ss0, ss1, ss2):
    def scale_rows(rowbuf, ebuf, eoff):
        @pl.loop(0, CH, step=8)
        def _(r):
            idx = jnp.full((16,), eoff, jnp.int32) + r
            sp = [plsc.load_gather(ebuf, [idx + k]) for k in range(8)]
            for cc in range(D // 16):
                sl = pl.ds(cc * 16, 16)
                for k in range(8):
                    rowbuf[r + k, sl] = rowbuf[r + k, sl] * sp[k]

    src2 = (srcA, srcB)
    dst2 = (dstA, dstB)
    e2 = (eA, eB)
    rows = (rows0, rows1, rows2)
    sidx = (sidx0, sidx1, sidx2)
    didx = (didx0, didx1, didx2)
    sg = (sg0, sg1, sg2)
    ss = (ss0, ss1, ss2)
    sst = (sst0, sst1)
    cid = lax.axis_index("c")
    sid = lax.axis_index("s")
    base = sid * NE

    pltpu.async_copy(m_hbm, m_v, sem).wait()
    mx = m_v[0]
    for i in range(1, NW):
        mx = jnp.maximum(mx, m_v[i])
    gmax = jnp.max(mx)

    @pl.loop(0, NP, step=16)
    def _(i):
        den_acc[pl.ds(i, 16)] = jnp.zeros((16,), jnp.float32)

    @pl.loop(0, CH)
    def _(r):
        for c in range(D // 16):
            rows0[r, pl.ds(c * 16, 16)] = jnp.zeros((16,), jnp.float32)

    @pl.loop(0, NR, step=CH)
    def _(i):
        pltpu.sync_copy(rows0, acc_sh.at[pl.ds(sid * NR + i, CH), :])

    plsc.subcore_barrier()

    def stage_group(g, p):
        off = base + g * GSZ
        pltpu.async_copy(src_hbm.at[pl.ds(off, GSZ)], src2[p], sst[p])
        pltpu.async_copy(dst_hbm.at[pl.ds(off, GSZ)], dst2[p], sst[p])
        pltpu.async_copy(e_hbm.at[pl.ds(off, GSZ)], e2[p], sst[p])

    def wait_stage(g, p):
        off = base + g * GSZ
        pltpu.make_async_copy(src_hbm.at[pl.ds(off, GSZ)], src2[p],
                              sst[p]).wait()
        pltpu.make_async_copy(dst_hbm.at[pl.ds(off, GSZ)], dst2[p],
                              sst[p]).wait()
        pltpu.make_async_copy(e_hbm.at[pl.ds(off, GSZ)], e2[p],
                              sst[p]).wait()

    def wait_scatter(b):
        pltpu.make_async_copy(rows[b], acc_sh.at[didx[b]], ss[b]).wait()

    def wait_gather(g, b):
        @pl.when(cid == 0)
        def _():
            pltpu.make_async_copy(xl_hbm.at[sidx[b]], rows[b], sg[b]).wait()

        @pl.when(cid != 0)
        def _():
            pltpu.make_async_copy(
                rh_hbm.at[pl.ds(base + (g * 3 + b) * CH, CH), :],
                rows[b], sg[b]).wait()

    def process_group(g, p, first):
        wait_stage(g, p)
        for b in range(3):
            if not first:
                wait_scatter(b)
            for j in range(CH // 16):
                didx[b][pl.ds(j * 16, 16)] = dst2[p][pl.ds(b * CH + j * 16, 16)]

            @pl.when(cid == 0)
            def _(b=b):
                for j in range(CH // 16):
                    sidx[b][pl.ds(j * 16, 16)] = src2[p][
                        pl.ds(b * CH + j * 16, 16)]
                pltpu.async_copy(xl_hbm.at[sidx[b]], rows[b], sg[b])

            @pl.when(cid != 0)
            def _(b=b):
                pltpu.async_copy(
                    rh_hbm.at[pl.ds(base + (g * 3 + b) * CH, CH), :],
                    rows[b], sg[b])

        for j in range(GSZ // 16):
            sl = pl.ds(j * 16, 16)
            e2[p][sl] = jnp.exp(e2[p][sl] - gmax)

        @pl.when(cid == 0)
        def _():
            for j in range(GSZ // 16):
                sl = pl.ds(j * 16, 16)
                plsc.addupdate_scatter(den_acc, [dst2[p][sl]], e2[p][sl])

        for b in range(3):
            wait_gather(g, b)
            scale_rows(rows[b], e2[p], b * CH)
            pltpu.async_copy(rows[b], acc_sh.at[didx[b]], ss[b], add=True)

        if isinstance(g, int):
            if g + 2 < NG:
                stage_group(g + 2, p)
        else:
            @pl.when(g + 2 < NG)
            def _():
                stage_group(g + 2, p)

    stage_group(0, 0)
    stage_group(1, 1)
    process_group(0, 0, first=True)
    process_group(1, 1, first=False)

    @pl.loop(1, NG // 2)
    def _(kk):
        process_group(2 * kk, 0, first=False)
        process_group(2 * kk + 1, 1, first=False)

    for b in range(3):
        wait_scatter(b)

    LO = NG * GSZ
    for t in range(NE // CH - NG * 3):
        off = base + LO + t * CH
        pltpu.async_copy(dst_hbm.at[pl.ds(off, CH)], didx0, sem).wait()
        pltpu.async_copy(e_hbm.at[pl.ds(off, CH)],
                         eA.at[pl.ds(0, CH)], sem).wait()

        @pl.when(cid == 0)
        def _(off=off):
            pltpu.async_copy(src_hbm.at[pl.ds(off, CH)], sidx0, sem).wait()
            pltpu.async_copy(xl_hbm.at[sidx0], rows0, sem).wait()

        @pl.when(cid != 0)
        def _(off=off):
            pltpu.async_copy(rh_hbm.at[pl.ds(off, CH), :], rows0, sem).wait()

        for j in range(CH // 16):
            sl = pl.ds(j * 16, 16)
            eA[sl] = jnp.exp(eA[sl] - gmax)

        @pl.when(cid == 0)
        def _():
            for j in range(CH // 16):
                sl = pl.ds(j * 16, 16)
                plsc.addupdate_scatter(den_acc, [didx0[sl]], eA[sl])

        scale_rows(rows0, eA, 0)
        pltpu.sync_copy(rows0, acc_sh.at[didx0], add=True)

    plsc.subcore_barrier()

    @pl.when(cid == 0)
    def _():
        pltpu.sync_copy(acc_sh.at[pl.ds(sid * NR, NR), :],
                        p_hbm.at[pl.ds(sid * NR, NR), :])

    @pl.when(cid != 0)
    def _():
        pltpu.sync_copy(acc_sh.at[pl.ds(sid * NR, NR), :],
                        q_hbm.at[pl.ds(sid * NR, NR), :])

    @pl.when(cid == 0)
    def _():
        pltpu.sync_copy(den_acc, den_hbm.at[sid])


def _segment_phase(st, u, src, dst, xl, r_h):
    sp = jnp.zeros((NP,), jnp.float32).at[:N].set(st[:, 0])
    tp = jnp.zeros((NP,), jnp.float32).at[:N].set(st[:, 1])
    e, m = _s1(sp, tp, src, dst, u)
    xlp = jnp.zeros((NP, D), jnp.float32).at[:N].set(xl)
    P, Q, den16 = _s2(e, m, src, dst, xlp, r_h)
    return P[:N], Q[:N], den16[:, :N]


def _layer(xl, r_h, src, dst, u, W, Wr, att, Lw):
    h, st = _pre(xl, W, att)
    P, Q, denom = _segment_phase(st, u, src, dst, xl, r_h)
    return _post(P, Q, h, denom, W, Wr, Lw)


def kernel(x, edge_index, r_h, W1, Wr1, att1, loop1, W2, Wr2, att2, loop2):
    src = edge_index[0]
    dst = edge_index[1]
    u = _u_pass(r_h, Wr1, att1, Wr2, att2)
    h1 = _layer(x, r_h, src, dst, u[:, 0], W1, Wr1, att1, loop1)
    h2 = _layer(h1, r_h, src, dst, u[:, 1], W2, Wr2, att2, loop2)
    return h2


# speedup vs baseline: 1.0128x; 1.0128x over previous
import dataclasses
import functools

import jax
import jax.numpy as jnp
from jax import lax
from jax.experimental import pallas as pl
from jax.experimental.pallas import tpu as pltpu
from jax.experimental.pallas import tpu_sc as plsc

N = 10000
E = 320000
D = 128
NP = 10240
NW = 32
EW = E // NW

_sc_mesh = plsc.VectorSubcoreMesh(core_axis_name="c", subcore_axis_name="s")

_sc_params = pltpu.CompilerParams()
if "needs_layout_passes" in pltpu.CompilerParams.__dataclass_fields__:
    _sc_params = dataclasses.replace(_sc_params, needs_layout_passes=False)


def _upre_body(wr1_ref, att1_ref, wr2_ref, att2_ref, w1_ref, xl_ref, rh_ref,
               u_ref, h_ref, st_ref):
    vr1 = jnp.dot(wr1_ref[...].T, att1_ref[0, 2 * D:],
                  preferred_element_type=jnp.float32)
    vr2 = jnp.dot(wr2_ref[...].T, att2_ref[0, 2 * D:],
                  preferred_element_type=jnp.float32)
    u_ref[...] = jnp.dot(rh_ref[...], jnp.stack([vr1, vr2], axis=1),
                         preferred_element_type=jnp.float32)

    @pl.when(pl.program_id(0) == 0)
    def _():
        w = w1_ref[...]
        att = att1_ref[...]
        vs = jnp.dot(w.T, att[0, :D], preferred_element_type=jnp.float32)
        vd = jnp.dot(w.T, att[0, D:2 * D], preferred_element_type=jnp.float32)
        xl = xl_ref[...]
        h_ref[...] = jnp.dot(xl, w.T, preferred_element_type=jnp.float32)
        st_ref[...] = jnp.dot(xl, jnp.stack([vs, vd], axis=1),
                              preferred_element_type=jnp.float32)


@jax.jit
def _u_pre(r_h, Wr1, att1, Wr2, att2, x, W1):
    blk = 16000
    return pl.pallas_call(
        _upre_body,
        grid=(E // blk,),
        in_specs=[
            pl.BlockSpec((D, D), lambda i: (0, 0)),
            pl.BlockSpec((1, 3 * D), lambda i: (0, 0)),
            pl.BlockSpec((D, D), lambda i: (0, 0)),
            pl.BlockSpec((1, 3 * D), lambda i: (0, 0)),
            pl.BlockSpec((D, D), lambda i: (0, 0)),
            pl.BlockSpec((N, D), lambda i: (0, 0)),
            pl.BlockSpec((blk, D), lambda i: (i, 0)),
        ],
        out_specs=[
            pl.BlockSpec((blk, 2), lambda i: (i, 0)),
            pl.BlockSpec((N, D), lambda i: (0, 0)),
            pl.BlockSpec((N, 2), lambda i: (0, 0)),
        ],
        out_shape=[jax.ShapeDtypeStruct((E, 2), jnp.float32),
                   jax.ShapeDtypeStruct((N, D), jnp.float32),
                   jax.ShapeDtypeStruct((N, 2), jnp.float32)],
    )(Wr1, att1, Wr2, att2, W1, x, r_h)


def _post_core(p_ref, q_ref, h_ref, den_ref, w_ref, wr_ref, lw_ref):
    den = jnp.sum(den_ref[...], axis=0)[:, None]
    safe = jnp.where(den > 0.0, den, 1.0)
    h = h_ref[...]
    agg = (jnp.dot(p_ref[...], w_ref[...].T, preferred_element_type=jnp.float32)
           + jnp.dot(q_ref[...], wr_ref[...].T, preferred_element_type=jnp.float32)) / safe
    out = jnp.where(den > 0.0,
                    agg + jnp.dot(h, lw_ref[...], preferred_element_type=jnp.float32),
                    h)
    return jnp.maximum(out, 0.0)


def _postpre_body(p_ref, q_ref, h_ref, den_ref, w_ref, wr_ref, lw_ref,
                  w2_ref, att2_ref, op_ref, h2_ref, st2_ref):
    out = _post_core(p_ref, q_ref, h_ref, den_ref, w_ref, wr_ref, lw_ref)
    op_ref[...] = jnp.concatenate(
        [out, jnp.zeros((NP - N, D), jnp.float32)], axis=0)
    w2 = w2_ref[...]
    att2 = att2_ref[...]
    vs = jnp.dot(w2.T, att2[0, :D], preferred_element_type=jnp.float32)
    vd = jnp.dot(w2.T, att2[0, D:2 * D], preferred_element_type=jnp.float32)
    h2_ref[...] = jnp.dot(out, w2.T, preferred_element_type=jnp.float32)
    st2_ref[...] = jnp.dot(out, jnp.stack([vs, vd], axis=1),
                           preferred_element_type=jnp.float32)


@jax.jit
def _postpre(P, Q, h, den16, W, Wr, Lw, W2, att2):
    return pl.pallas_call(
        _postpre_body,
        out_shape=[jax.ShapeDtypeStruct((NP, D), jnp.float32),
                   jax.ShapeDtypeStruct((N, D), jnp.float32),
                   jax.ShapeDtypeStruct((N, 2), jnp.float32)],
    )(P, Q, h, den16, W, Wr, Lw, W2, att2)


def _post_body(p_ref, q_ref, h_ref, den_ref, w_ref, wr_ref, lw_ref, o_ref):
    o_ref[...] = _post_core(p_ref, q_ref, h_ref, den_ref, w_ref, wr_ref,
                            lw_ref)


@jax.jit
def _post(P, Q, h, den16, W, Wr, Lw):
    return pl.pallas_call(
        _post_body,
        out_shape=jax.ShapeDtypeStruct((N, D), jnp.float32),
    )(P, Q, h, den16, W, Wr, Lw)


@functools.partial(
    pl.kernel,
    out_type=[jax.ShapeDtypeStruct((E,), jnp.float32),
              jax.ShapeDtypeStruct((NW, 16), jnp.float32)],
    mesh=_sc_mesh,
    scratch_types=[
        pltpu.VMEM((NP,), jnp.float32),
        pltpu.VMEM((NP,), jnp.float32),
        pltpu.VMEM((EW,), jnp.int32),
        pltpu.VMEM((EW,), jnp.int32),
        pltpu.VMEM((EW,), jnp.float32),
        pltpu.VMEM((EW,), jnp.float32),
        pltpu.VMEM((16,), jnp.float32),
        pltpu.SemaphoreType.DMA,
    ],
    compiler_params=_sc_params)
def _s1(s_hbm, t_hbm, src_hbm, dst_hbm, u_hbm, e_hbm, m_hbm,
        s_v, t_v, src_v, dst_v, u_v, e_v, mx_v, sem):
    wid = lax.axis_index("s") * 2 + lax.axis_index("c")
    base = wid * EW
    pltpu.async_copy(s_hbm, s_v, sem).wait()
    pltpu.async_copy(t_hbm, t_v, sem).wait()
    pltpu.async_copy(src_hbm.at[pl.ds(base, EW)], src_v, sem).wait()
    pltpu.async_copy(dst_hbm.at[pl.ds(base, EW)], dst_v, sem).wait()
    pltpu.async_copy(u_hbm.at[pl.ds(base, EW)], u_v, sem).wait()
    mx_v[...] = jnp.full((16,), -3.0e38, jnp.float32)

    @pl.loop(0, EW, step=16)
    def _(i):
        sl = pl.ds(i, 16)
        e16 = (plsc.load_gather(s_v, [src_v[sl]])
               + plsc.load_gather(t_v, [dst_v[sl]])
               + u_v[sl])
        e16 = jnp.where(e16 >= 0.0, e16, 0.01 * e16)
        e_v[sl] = e16
        mx_v[...] = jnp.maximum(mx_v[...], e16)

    pltpu.async_copy(e_v, e_hbm.at[pl.ds(base, EW)], sem).wait()
    pltpu.async_copy(mx_v, m_hbm.at[wid], sem).wait()


NT = 16
NE = E // NT
CH = 80
NR = NP // NT
NCH = NE // CH
GSZ = 3 * CH
NG = 82


@functools.partial(
    pl.kernel,
    out_type=[jax.ShapeDtypeStruct((NP, D), jnp.float32),
              jax.ShapeDtypeStruct((NP, D), jnp.float32),
              jax.ShapeDtypeStruct((NT, NP), jnp.float32)],
    mesh=_sc_mesh,
    scratch_types=(
        [pltpu.VMEM_SHARED((NP, D), jnp.float32)]
        + [pltpu.VMEM((GSZ,), jnp.int32) for _ in range(2)]
        + [pltpu.VMEM((GSZ,), jnp.int32) for _ in range(2)]
        + [pltpu.VMEM((GSZ,), jnp.float32) for _ in range(2)]
        + [pltpu.VMEM((CH, D), jnp.float32) for _ in range(3)]
        + [pltpu.VMEM((CH,), jnp.int32) for _ in range(3)]
        + [pltpu.VMEM((CH,), jnp.int32) for _ in range(3)]
        + [pltpu.VMEM((NP,), jnp.float32),
           pltpu.VMEM((NW, 16), jnp.float32)]
        + [pltpu.SemaphoreType.DMA]
        + [pltpu.SemaphoreType.DMA for _ in range(2)]
        + [pltpu.SemaphoreType.DMA for _ in range(6)]
    ),
    compiler_params=_sc_params)
def _s2(e_hbm, m_hbm, src_hbm, dst_hbm, xl_hbm, rh_hbm,
        p_hbm, q_hbm, den_hbm,
        acc_sh, srcA, srcB, dstA, dstB, eA, eB,
        rows0, rows1, rows2,
        sidx0, sidx1, sidx2,
        didx0, didx1, didx2,
        den_acc, m_v, sem, sst0, sst1,
        sg0, sg1, sg2, ss0, ss1, ss2):
    def scale_rows(rowbuf, ebuf, eoff):
        @pl.loop(0, CH, step=4)
        def _(r):
            idx = jnp.full((16,), eoff, jnp.int32) + r
            sp = [plsc.load_gather(ebuf, [idx + k]) for k in range(4)]
            for cc in range(D // 16):
                sl = pl.ds(cc * 16, 16)
                for k in range(4):
                    rowbuf[r + k, sl] = rowbuf[r + k, sl] * sp[k]

    src2 = (srcA, srcB)
    dst2 = (dstA, dstB)
    e2 = (eA, eB)
    rows = (rows0, rows1, rows2)
    sidx = (sidx0, sidx1, sidx2)
    didx = (didx0, didx1, didx2)
    sg = (sg0, sg1, sg2)
    ss = (ss0, ss1, ss2)
    sst = (sst0, sst1)
    cid = lax.axis_index("c")
    sid = lax.axis_index("s")
    base = sid * NE

    pltpu.async_copy(m_hbm, m_v, sem).wait()
    mx = m_v[0]
    for i in range(1, NW):
        mx = jnp.maximum(mx, m_v[i])
    gmax = jnp.max(mx)

    @pl.loop(0, NP, step=16)
    def _(i):
        den_acc[pl.ds(i, 16)] = jnp.zeros((16,), jnp.float32)

    @pl.loop(0, CH)
    def _(r):
        for c in range(D // 16):
            rows0[r, pl.ds(c * 16, 16)] = jnp.zeros((16,), jnp.float32)

    @pl.loop(0, NR, step=CH)
    def _(i):
        pltpu.sync_copy(rows0, acc_sh.at[pl.ds(sid * NR + i, CH), :])

    plsc.subcore_barrier()

    def stage_group(g, p):
        off = base + g * GSZ
        pltpu.async_copy(src_hbm.at[pl.ds(off, GSZ)], src2[p], sst[p])
        pltpu.async_copy(dst_hbm.at[pl.ds(off, GSZ)], dst2[p], sst[p])
        pltpu.async_copy(e_hbm.at[pl.ds(off, GSZ)], e2[p], sst[p])

    def wait_stage(g, p):
        off = base + g * GSZ
        pltpu.make_async_copy(src_hbm.at[pl.ds(off, GSZ)], src2[p],
                              sst[p]).wait()
        pltpu.make_async_copy(dst_hbm.at[pl.ds(off, GSZ)], dst2[p],
                              sst[p]).wait()
        pltpu.make_async_copy(e_hbm.at[pl.ds(off, GSZ)], e2[p],
                              sst[p]).wait()

    def wait_scatter(b):
        pltpu.make_async_copy(rows[b], acc_sh.at[didx[b]], ss[b]).wait()

    def wait_gather(g, b):
        @pl.when(cid == 0)
        def _():
            pltpu.make_async_copy(xl_hbm.at[sidx[b]], rows[b], sg[b]).wait()

        @pl.when(cid != 0)
        def _():
            pltpu.make_async_copy(
                rh_hbm.at[pl.ds(base + (g * 3 + b) * CH, CH), :],
                rows[b], sg[b]).wait()

    def process_group(g, p, first):
        wait_stage(g, p)
        for b in range(3):
            if not first:
                wait_scatter(b)
            for j in range(CH // 16):
                didx[b][pl.ds(j * 16, 16)] = dst2[p][pl.ds(b * CH + j * 16, 16)]

            @pl.when(cid == 0)
            def _(b=b):
                for j in range(CH // 16):
                    sidx[b][pl.ds(j * 16, 16)] = src2[p][
                        pl.ds(b * CH + j * 16, 16)]
                pltpu.async_copy(xl_hbm.at[sidx[b]], rows[b], sg[b])

            @pl.when(cid != 0)
            def _(b=b):
                pltpu.async_copy(
                    rh_hbm.at[pl.ds(base + (g * 3 + b) * CH, CH), :],
                    rows[b], sg[b])

        for j in range(GSZ // 16):
            sl = pl.ds(j * 16, 16)
            e2[p][sl] = jnp.exp(e2[p][sl] - gmax)

        @pl.when(cid == 0)
        def _():
            for j in range(GSZ // 16):
                sl = pl.ds(j * 16, 16)
                plsc.addupdate_scatter(den_acc, [dst2[p][sl]], e2[p][sl])

        for b in range(3):
            wait_gather(g, b)
            scale_rows(rows[b], e2[p], b * CH)
            pltpu.async_copy(rows[b], acc_sh.at[didx[b]], ss[b], add=True)

        if isinstance(g, int):
            if g + 2 < NG:
                stage_group(g + 2, p)
        else:
            @pl.when(g + 2 < NG)
            def _():
                stage_group(g + 2, p)

    stage_group(0, 0)
    stage_group(1, 1)
    process_group(0, 0, first=True)
    process_group(1, 1, first=False)

    @pl.loop(1, NG // 2)
    def _(kk):
        process_group(2 * kk, 0, first=False)
        process_group(2 * kk + 1, 1, first=False)

    for b in range(3):
        wait_scatter(b)

    LO = NG * GSZ
    for t in range(NE // CH - NG * 3):
        off = base + LO + t * CH
        pltpu.async_copy(dst_hbm.at[pl.ds(off, CH)], didx0, sem).wait()
        pltpu.async_copy(e_hbm.at[pl.ds(off, CH)],
                         eA.at[pl.ds(0, CH)], sem).wait()

        @pl.when(cid == 0)
        def _(off=off):
            pltpu.async_copy(src_hbm.at[pl.ds(off, CH)], sidx0, sem).wait()
            pltpu.async_copy(xl_hbm.at[sidx0], rows0, sem).wait()

        @pl.when(cid != 0)
        def _(off=off):
            pltpu.async_copy(rh_hbm.at[pl.ds(off, CH), :], rows0, sem).wait()

        for j in range(CH // 16):
            sl = pl.ds(j * 16, 16)
            eA[sl] = jnp.exp(eA[sl] - gmax)

        @pl.when(cid == 0)
        def _():
            for j in range(CH // 16):
                sl = pl.ds(j * 16, 16)
                plsc.addupdate_scatter(den_acc, [didx0[sl]], eA[sl])

        scale_rows(rows0, eA, 0)
        pltpu.sync_copy(rows0, acc_sh.at[didx0], add=True)

    plsc.subcore_barrier()

    @pl.when(cid == 0)
    def _():
        pltpu.sync_copy(acc_sh.at[pl.ds(sid * NR, NR), :],
                        p_hbm.at[pl.ds(sid * NR, NR), :])

    @pl.when(cid != 0)
    def _():
        pltpu.sync_copy(acc_sh.at[pl.ds(sid * NR, NR), :],
                        q_hbm.at[pl.ds(sid * NR, NR), :])

    @pl.when(cid == 0)
    def _():
        pltpu.sync_copy(den_acc, den_hbm.at[sid])


def _segment_phase(st, u, src, dst, xlp, r_h):
    sp = jnp.zeros((NP,), jnp.float32).at[:N].set(st[:, 0])
    tp = jnp.zeros((NP,), jnp.float32).at[:N].set(st[:, 1])
    e, m = _s1(sp, tp, src, dst, u)
    P, Q, den16 = _s2(e, m, src, dst, xlp, r_h)
    return P[:N], Q[:N], den16[:, :N]


def kernel(x, edge_index, r_h, W1, Wr1, att1, loop1, W2, Wr2, att2, loop2):
    src = edge_index[0]
    dst = edge_index[1]
    u, h1, st1 = _u_pre(r_h, Wr1, att1, Wr2, att2, x, W1)
    x1p = jnp.zeros((NP, D), jnp.float32).at[:N].set(x)
    P1, Q1, d1 = _segment_phase(st1, u[:, 0], src, dst, x1p, r_h)
    x2p, h2, st2 = _postpre(P1, Q1, h1, d1, W1, Wr1, loop1, W2, att2)
    P2, Q2, d2 = _segment_phase(st2, u[:, 1], src, dst, x2p, r_h)
    return _post(P2, Q2, h2, d2, W2, Wr2, loop2)
